# baseline (speedup 1.0000x reference)
import functools

import jax
import jax.numpy as jnp
from jax import lax
from jax.experimental import pallas as pl
from jax.experimental.pallas import tpu as pltpu
from jax.experimental.pallas import tpu_sc as plsc

N, S, D = 1, 2048, 768
OUT = 768
E = 8
TOP_K = 2

TILE_R = 128
NTILES = 40
NSLOT = NTILES * TILE_R
EPR_PAD = 64

NW = 32
TPW = S // NW


def _route_kernel(x_ref, wg_ref, bg_ref, pos_ref, epr_ref):
    x = x_ref[...]
    logits = lax.dot_general(
        x, wg_ref[...], (((1,), (1,)), ((), ())),
        preferred_element_type=jnp.float32)
    logits = logits + bg_ref[...]

    lane = lax.broadcasted_iota(jnp.int32, (S, E), 1)
    big = jnp.int32(E)
    m1 = jnp.max(logits, axis=1, keepdims=True)
    a1 = jnp.min(jnp.where(logits == m1, lane, big), axis=1, keepdims=True)
    logits2 = jnp.where(lane == a1, jnp.float32(-jnp.inf), logits)
    m2 = jnp.max(logits2, axis=1, keepdims=True)
    a2 = jnp.min(jnp.where(logits2 == m2, lane, big), axis=1, keepdims=True)
    mask = ((lane == a1) | (lane == a2)).astype(jnp.float32)

    c = mask
    sh = 1
    while sh < S:
        shifted = jnp.concatenate(
            [jnp.zeros((sh, E), jnp.float32), c[: S - sh]], axis=0)
        c = c + shifted
        sh *= 2
    cx = c - mask
    counts = c[S - 1:S, :]

    seg = jnp.ceil(counts * (1.0 / TILE_R)) * float(TILE_R)
    aseg = seg
    sh = 1
    while sh < E:
        shifted = jnp.concatenate(
            [jnp.zeros((1, sh), jnp.float32), aseg[:, : E - sh]], axis=1)
        aseg = aseg + shifted
        sh *= 2
    astart = aseg - seg

    posmat = astart + cx
    sel0 = jnp.sum(jnp.where(lane == a1, posmat, 0.0), axis=1, keepdims=True)
    sel1 = jnp.sum(jnp.where(lane == a2, posmat, 0.0), axis=1, keepdims=True)
    pos = jnp.where(lane == 0, sel0, jnp.where(lane == 1, sel1, 0.0))
    pos_ref[...] = pos.astype(jnp.int32)

    tile_start = lax.broadcasted_iota(
        jnp.int32, (E, EPR_PAD), 1).astype(jnp.float32) * float(TILE_R)
    astart_col = jnp.broadcast_to(
        astart.reshape(E, 1), (E, EPR_PAD))
    ge = (tile_start >= astart_col).astype(jnp.float32)
    epr = jnp.sum(ge, axis=0, keepdims=True) - 1.0
    epr = jnp.clip(epr, 0.0, float(E - 1))
    epr_ref[...] = epr.astype(jnp.int32)


def _route(x2, Wg, bg2):
    return pl.pallas_call(
        _route_kernel,
        in_specs=[
            pl.BlockSpec((S, D), lambda: (0, 0)),
            pl.BlockSpec((E, D), lambda: (0, 0)),
            pl.BlockSpec((1, E), lambda: (0, 0)),
        ],
        out_specs=[
            pl.BlockSpec((S, E), lambda: (0, 0)),
            pl.BlockSpec((1, EPR_PAD), lambda: (0, 0)),
        ],
        out_shape=[
            jax.ShapeDtypeStruct((S, E), jnp.int32),
            jax.ShapeDtypeStruct((1, EPR_PAD), jnp.int32),
        ],
    )(x2, Wg, bg2)


def _scatter_body(x_hbm, post_hbm, xg_hbm, idx0_v, idx1_v, rows_v, sem):
    wid = lax.axis_index("s") * 2 + lax.axis_index("c")
    base = wid * TPW
    pltpu.sync_copy(post_hbm.at[0, pl.ds(base, TPW)], idx0_v)
    pltpu.sync_copy(post_hbm.at[1, pl.ds(base, TPW)], idx1_v)
    pltpu.sync_copy(x_hbm.at[pl.ds(base, TPW)], rows_v)
    c0 = pltpu.async_copy(rows_v, xg_hbm.at[idx0_v], sem)
    c1 = pltpu.async_copy(rows_v, xg_hbm.at[idx1_v], sem)
    c0.wait()
    c1.wait()


def _scatter(x2, posT):
    mesh = plsc.VectorSubcoreMesh(core_axis_name="c", subcore_axis_name="s")
    f = functools.partial(
        pl.kernel,
        mesh=mesh,
        out_type=jax.ShapeDtypeStruct((NSLOT, D), jnp.float32),
        scratch_types=[
            pltpu.VMEM((TPW,), jnp.int32),
            pltpu.VMEM((TPW,), jnp.int32),
            pltpu.VMEM((TPW, D), jnp.float32),
            pltpu.SemaphoreType.DMA,
        ],
    )(_scatter_body)
    return f(x2, posT)


def _mm_kernel(epr_sref, xg_ref, we_ref, be_ref, y_ref):
    acc = lax.dot_general(
        xg_ref[...], we_ref[0], (((1,), (0,)), ((), ())),
        preferred_element_type=jnp.float32)
    y_ref[...] = jnp.maximum(acc + be_ref[0], 0.0)


def _grouped_mm(xg, We, be, epr):
    grid_spec = pltpu.PrefetchScalarGridSpec(
        num_scalar_prefetch=1,
        grid=(NTILES,),
        in_specs=[
            pl.BlockSpec((TILE_R, D), lambda i, epr_s: (i, 0)),
            pl.BlockSpec((1, D, OUT), lambda i, epr_s: (epr_s[i], 0, 0)),
            pl.BlockSpec((1, 1, OUT), lambda i, epr_s: (epr_s[i], 0, 0)),
        ],
        out_specs=pl.BlockSpec((TILE_R, OUT), lambda i, epr_s: (i, 0)),
    )
    return pl.pallas_call(
        _mm_kernel,
        grid_spec=grid_spec,
        out_shape=jax.ShapeDtypeStruct((NSLOT, OUT), jnp.float32),
    )(epr, xg, We, be.reshape(E, 1, OUT))


def _combine_body(y_hbm, post_hbm, out_hbm, idx0_v, idx1_v, r0_v, r1_v, sem):
    wid = lax.axis_index("s") * 2 + lax.axis_index("c")
    base = wid * TPW
    pltpu.sync_copy(post_hbm.at[0, pl.ds(base, TPW)], idx0_v)
    pltpu.sync_copy(post_hbm.at[1, pl.ds(base, TPW)], idx1_v)
    c0 = pltpu.async_copy(y_hbm.at[idx0_v], r0_v, sem)
    c1 = pltpu.async_copy(y_hbm.at[idx1_v], r1_v, sem)
    c0.wait()
    c1.wait()

    def body(j, _):
        for cchunk in range(OUT // 16):
            sl = pl.ds(cchunk * 16, 16)
            r0_v[j, sl] = (r0_v[j, sl] + r1_v[j, sl]) * jnp.float32(0.5)
        return 0

    lax.fori_loop(0, TPW, body, 0)
    pltpu.sync_copy(r0_v, out_hbm.at[pl.ds(base, TPW)])


def _combine(y, posT):
    mesh = plsc.VectorSubcoreMesh(core_axis_name="c", subcore_axis_name="s")
    f = functools.partial(
        pl.kernel,
        mesh=mesh,
        out_type=jax.ShapeDtypeStruct((S, OUT), jnp.float32),
        scratch_types=[
            pltpu.VMEM((TPW,), jnp.int32),
            pltpu.VMEM((TPW,), jnp.int32),
            pltpu.VMEM((TPW, OUT), jnp.float32),
            pltpu.VMEM((TPW, OUT), jnp.float32),
            pltpu.SemaphoreType.DMA,
        ],
    )(_combine_body)
    return f(y, posT)


def kernel(x, Wg, bg, We, be):
    x2 = x.reshape(S, D)
    bg2 = bg.reshape(1, E)
    pos, epr = _route(x2, Wg, bg2)
    posT = jnp.transpose(pos[:, :TOP_K])
    xg = _scatter(x2, posT)
    y = _grouped_mm(xg, We, be, epr.reshape(EPR_PAD))
    out = _combine(y, posT)
    return out.reshape(N, S, OUT)


# speedup vs baseline: 2.1002x; 2.1002x over previous
import functools

import jax
import jax.numpy as jnp
from jax.experimental import pallas as pl
from jax.experimental.pallas import tpu as pltpu

N, S, D = 1, 2048, 768
OUT = 768
E = 8
TOP_K = 2

TILE_S = 512


def _moe_dense_kernel(x_ref, wg_ref, bg_ref, we_hbm, be_ref, out_ref,
                      we_vmem, sems):
    i = pl.program_id(0)

    @pl.when(i == 0)
    def _start_we_copies():
        for e in range(E):
            pltpu.make_async_copy(
                we_hbm.at[e], we_vmem.at[e], sems.at[e]).start()

    x = x_ref[...]
    logits = jax.lax.dot_general(
        x, wg_ref[...], (((1,), (1,)), ((), ())),
        preferred_element_type=jnp.float32)
    logits = logits + bg_ref[...]

    lane = jax.lax.broadcasted_iota(jnp.int32, (TILE_S, E), 1)
    big = jnp.int32(E)
    m1 = jnp.max(logits, axis=1, keepdims=True)
    a1 = jnp.min(jnp.where(logits == m1, lane, big), axis=1, keepdims=True)
    neg = jnp.float32(-jnp.inf)
    logits2 = jnp.where(lane == a1, neg, logits)
    m2 = jnp.max(logits2, axis=1, keepdims=True)
    a2 = jnp.min(jnp.where(logits2 == m2, lane, big), axis=1, keepdims=True)
    mask = ((lane == a1) | (lane == a2)).astype(jnp.float32)

    acc = jnp.zeros((TILE_S, OUT), dtype=jnp.float32)
    for e in range(E):
        @pl.when(i == 0)
        def _wait_we():
            pltpu.make_async_copy(
                we_hbm.at[e], we_vmem.at[e], sems.at[e]).wait()

        y = jax.lax.dot_general(
            x, we_vmem[e], (((1,), (0,)), ((), ())),
            preferred_element_type=jnp.float32)
        y = jnp.maximum(y + be_ref[e][None, :], 0.0)
        acc = acc + mask[:, e][:, None] * y
    out_ref[...] = acc * jnp.float32(1.0 / TOP_K)


def kernel(x, Wg, bg, We, be):
    x2 = x.reshape(S, D)
    bg2 = bg.reshape(1, E)
    grid = (S // TILE_S,)
    out = pl.pallas_call(
        _moe_dense_kernel,
        grid=grid,
        in_specs=[
            pl.BlockSpec((TILE_S, D), lambda i: (i, 0)),
            pl.BlockSpec((E, D), lambda i: (0, 0)),
            pl.BlockSpec((1, E), lambda i: (0, 0)),
            pl.BlockSpec(memory_space=pltpu.HBM),
            pl.BlockSpec((E, OUT), lambda i: (0, 0)),
        ],
        out_specs=pl.BlockSpec((TILE_S, OUT), lambda i: (i, 0)),
        out_shape=jax.ShapeDtypeStruct((S, OUT), jnp.float32),
        scratch_shapes=[
            pltpu.VMEM((E, D, OUT), jnp.float32),
            pltpu.SemaphoreType.DMA((E,)),
        ],
    )(x2, Wg, bg2, We, be)
    return out.reshape(N, S, OUT)


# speedup vs baseline: 2.7613x; 1.3148x over previous
import functools

import jax
import jax.numpy as jnp
from jax.experimental import pallas as pl
from jax.experimental.pallas import tpu as pltpu

N, S, D = 1, 2048, 768
OUT = 768
E = 8
TOP_K = 2

TILE_S = 512


def _moe_dense_kernel(x_ref, wg_ref, bg_ref, we_ref, be_ref, out_ref):
    x = x_ref[...]
    logits = jax.lax.dot_general(
        x, wg_ref[...], (((1,), (1,)), ((), ())),
        preferred_element_type=jnp.float32)
    logits = logits + bg_ref[...]

    lane = jax.lax.broadcasted_iota(jnp.int32, (TILE_S, E), 1)
    big = jnp.int32(E)
    m1 = jnp.max(logits, axis=1, keepdims=True)
    a1 = jnp.min(jnp.where(logits == m1, lane, big), axis=1, keepdims=True)
    neg = jnp.float32(-jnp.inf)
    logits2 = jnp.where(lane == a1, neg, logits)
    m2 = jnp.max(logits2, axis=1, keepdims=True)
    a2 = jnp.min(jnp.where(logits2 == m2, lane, big), axis=1, keepdims=True)
    mask = ((lane == a1) | (lane == a2)).astype(jnp.float32)

    acc = jnp.zeros((TILE_S, OUT), dtype=jnp.float32)
    for e in range(E):
        y = jax.lax.dot_general(
            x, we_ref[e], (((1,), (0,)), ((), ())),
            preferred_element_type=jnp.float32)
        y = jnp.maximum(y + be_ref[e][None, :], 0.0)
        acc = acc + mask[:, e][:, None] * y
    out_ref[...] = acc * jnp.float32(1.0 / TOP_K)


def kernel(x, Wg, bg, We, be):
    x2 = x.reshape(S, D)
    bg2 = bg.reshape(1, E)
    grid = (S // TILE_S,)
    out = pl.pallas_call(
        _moe_dense_kernel,
        grid=grid,
        in_specs=[
            pl.BlockSpec((TILE_S, D), lambda i: (i, 0)),
            pl.BlockSpec((E, D), lambda i: (0, 0)),
            pl.BlockSpec((1, E), lambda i: (0, 0)),
            pl.BlockSpec((E, D, OUT), lambda i: (0, 0, 0)),
            pl.BlockSpec((E, OUT), lambda i: (0, 0)),
        ],
        out_specs=pl.BlockSpec((TILE_S, OUT), lambda i: (i, 0)),
        out_shape=jax.ShapeDtypeStruct((S, OUT), jnp.float32),
    )(x2, Wg, bg2, We, be)
    return out.reshape(N, S, OUT)


# speedup vs baseline: 2.7669x; 1.0020x over previous
import functools

import jax
import jax.numpy as jnp
from jax.experimental import pallas as pl
from jax.experimental.pallas import tpu as pltpu

N, S, D = 1, 2048, 768
OUT = 768
E = 8
TOP_K = 2

TILE_S = 1024


def _moe_dense_kernel(x_ref, wg_ref, bg_ref, we_ref, be_ref, out_ref):
    x = x_ref[...]
    logits = jax.lax.dot_general(
        x, wg_ref[...], (((1,), (1,)), ((), ())),
        preferred_element_type=jnp.float32)
    logits = logits + bg_ref[...]

    lane = jax.lax.broadcasted_iota(jnp.int32, (TILE_S, E), 1)
    big = jnp.int32(E)
    m1 = jnp.max(logits, axis=1, keepdims=True)
    a1 = jnp.min(jnp.where(logits == m1, lane, big), axis=1, keepdims=True)
    neg = jnp.float32(-jnp.inf)
    logits2 = jnp.where(lane == a1, neg, logits)
    m2 = jnp.max(logits2, axis=1, keepdims=True)
    a2 = jnp.min(jnp.where(logits2 == m2, lane, big), axis=1, keepdims=True)
    mask = ((lane == a1) | (lane == a2)).astype(jnp.float32)

    acc = jnp.zeros((TILE_S, OUT), dtype=jnp.float32)
    for e in range(E):
        y = jax.lax.dot_general(
            x, we_ref[e], (((1,), (0,)), ((), ())),
            preferred_element_type=jnp.float32)
        y = jnp.maximum(y + be_ref[e][None, :], 0.0)
        acc = acc + mask[:, e][:, None] * y
    out_ref[...] = acc * jnp.float32(1.0 / TOP_K)


def kernel(x, Wg, bg, We, be):
    x2 = x.reshape(S, D)
    bg2 = bg.reshape(1, E)
    grid = (S // TILE_S,)
    out = pl.pallas_call(
        _moe_dense_kernel,
        grid=grid,
        in_specs=[
            pl.BlockSpec((TILE_S, D), lambda i: (i, 0)),
            pl.BlockSpec((E, D), lambda i: (0, 0)),
            pl.BlockSpec((1, E), lambda i: (0, 0)),
            pl.BlockSpec((E, D, OUT), lambda i: (0, 0, 0)),
            pl.BlockSpec((E, OUT), lambda i: (0, 0)),
        ],
        out_specs=pl.BlockSpec((TILE_S, OUT), lambda i: (i, 0)),
        out_shape=jax.ShapeDtypeStruct((S, OUT), jnp.float32),
    )(x2, Wg, bg2, We, be)
    return out.reshape(N, S, OUT)
